# baseline (speedup 1.0000x reference)
import functools

import jax
import jax.numpy as jnp
from jax import lax
from jax.experimental import pallas as pl
from jax.experimental.pallas import tpu as pltpu
from jax.experimental.pallas import tpu_sc as plsc

_B = 16384
_VOCAB_F1 = 3
_VOCAB_F2 = 11
_EMB_DIM = 10
_H1 = 20
_NCOMBO = _VOCAB_F1 * _VOCAB_F2
_TBL = 48

_NC, _NS, _L = 2, 16, 16
_NW = _NC * _NS
_BPW = _B // _NW

_OFF_EMB = 0
_OFF_W1 = 112
_OFF_B1 = 528
_OFF_W2 = 560
_OFF_B2 = 592
_WLEN = 608


@functools.partial(
    pl.kernel,
    out_type=jax.ShapeDtypeStruct((_B,), jnp.float32),
    mesh=plsc.VectorSubcoreMesh(core_axis_name="c", subcore_axis_name="s"),
    compiler_params=pltpu.CompilerParams(needs_layout_passes=False),
    scratch_types=[
        pltpu.VMEM((_BPW,), jnp.int32),
        pltpu.VMEM((_BPW,), jnp.int32),
        pltpu.VMEM((_WLEN,), jnp.float32),
        pltpu.VMEM((_H1 * _L,), jnp.float32),
        pltpu.VMEM((_TBL,), jnp.float32),
        pltpu.VMEM((_BPW,), jnp.float32),
        pltpu.SemaphoreType.DMA,
    ],
)
def _sc_kernel(f1_hbm, f2_hbm, wts_hbm, out_hbm,
               f1_v, f2_v, wts_v, m_v, tbl_v, out_v, sem):
    def splat(ref, off):
        return plsc.load_gather(ref, [jnp.full((_L,), off, jnp.int32)])

    wid = lax.axis_index("s") * _NC + lax.axis_index("c")
    base = wid * _BPW
    cw = pltpu.async_copy(wts_hbm, wts_v, sem)
    c1 = pltpu.async_copy(f1_hbm.at[pl.ds(base, _BPW)], f1_v, sem)
    c2 = pltpu.async_copy(f2_hbm.at[pl.ds(base, _BPW)], f2_v, sem)
    cw.wait()

    lanes = lax.iota(jnp.int32, _L)

    emb_cols = [plsc.load_gather(wts_v, [lanes * _EMB_DIM + k])
                for k in range(_EMB_DIM)]
    for j in range(_H1):
        acc = jnp.zeros((_L,), jnp.float32)
        for k in range(_EMB_DIM):
            acc = acc + emb_cols[k] * splat(wts_v, _OFF_W1 + (_VOCAB_F1 + k) * 32 + j)
        m_v[pl.ds(j * _L, _L)] = acc

    for t in range(3):
        c = lanes + t * _L
        a = c // _VOCAB_F2
        b = c - a * _VOCAB_F2
        acc = jnp.zeros((_L,), jnp.float32)
        for j in range(_H1):
            w1aj = plsc.load_gather(wts_v, [a * 32 + (_OFF_W1 + j)])
            mbj = plsc.load_gather(m_v, [b + j * _L])
            h = jnp.maximum(w1aj + mbj + splat(wts_v, _OFF_B1 + j), 0.0)
            acc = acc + h * splat(wts_v, _OFF_W2 + j)
        y = acc + splat(wts_v, _OFF_B2)
        tbl_v[pl.ds(t * _L, _L)] = y / (1.0 + jnp.abs(y))

    c1.wait()
    c2.wait()
    for i in range(_BPW // _L):
        s = pl.ds(i * _L, _L)
        idx = f1_v[s] * _VOCAB_F2 + f2_v[s]
        out_v[s] = plsc.load_gather(tbl_v, [idx])
    pltpu.sync_copy(out_v, out_hbm.at[pl.ds(base, _BPW)])


def kernel(f1, f2, emb_f2, W1, b1, W2, b2):
    f1 = f1.astype(jnp.int32)
    f2 = f2.astype(jnp.int32)
    wts = jnp.zeros((_WLEN,), jnp.float32)
    wts = wts.at[_OFF_EMB:_OFF_EMB + _VOCAB_F2 * _EMB_DIM].set(emb_f2.reshape(-1))
    w1p = jnp.pad(W1, ((0, 0), (0, 32 - _H1)))
    wts = wts.at[_OFF_W1:_OFF_W1 + 13 * 32].set(w1p.reshape(-1))
    wts = wts.at[_OFF_B1:_OFF_B1 + _H1].set(b1)
    wts = wts.at[_OFF_W2:_OFF_W2 + _H1].set(W2.reshape(-1))
    wts = wts.at[_OFF_B2].set(b2[0])
    out = _sc_kernel(f1, f2, wts)
    return out.reshape(_B, 1)
